# baseline (speedup 1.0000x reference)
import jax
import jax.numpy as jnp
from jax import lax
from jax.experimental import pallas as pl
from jax.experimental.pallas import tpu as pltpu

_VOCAB = 1_000_000
_D = 64
_NIDX = 20

_BV = 65536
_NB = (_VOCAB + _BV - 1) // _BV
_NEG = -1e30


def _fused_body(rows_ref, w_ref, b_ref, out_ref, scratch, s_ref, m_ref, l_ref):
  p = pl.program_id(0)
  i = pl.program_id(1)

  @pl.when(jnp.logical_and(p == 0, i == 0))
  def _():
    s_ref[...] = jnp.sum(rows_ref[...], axis=0, keepdims=True)
    m_ref[...] = jnp.full((1, 128), _NEG, jnp.float32)
    l_ref[...] = jnp.zeros((1, 128), jnp.float32)

  @pl.when(p == 0)
  def _():
    logits = lax.dot_general(
        s_ref[...], w_ref[0], (((1,), (0,)), ((), ())),
        preferred_element_type=jnp.float32,
    ) + b_ref[...].reshape(1, _BV)
    vidx = lax.broadcasted_iota(jnp.int32, (1, _BV), 1) + i * _BV
    logits = jnp.where(vidx < _VOCAB, logits, _NEG)
    scratch[:, pl.ds(i * _BV, _BV)] = logits
    m_old = m_ref[...]
    bmax = jnp.max(logits, axis=1, keepdims=True)
    m_new = jnp.maximum(m_old, bmax)
    corr = jnp.exp(m_old - m_new)
    bsum = jnp.sum(jnp.exp(logits - m_new[:, 0:1]), axis=1, keepdims=True)
    l_ref[...] = l_ref[...] * corr + bsum
    m_ref[...] = m_new

  @pl.when(p == 1)
  def _():
    logz = m_ref[...] + jnp.log(l_ref[...])
    res = scratch[:, pl.ds(i * _BV, _BV)] - logz[:, 0:1]
    out_ref[...] = res.reshape(_BV)


def kernel(inputs, emb, W, b):
  idx = inputs.astype(jnp.int32)
  rows = jnp.concatenate(
      [lax.dynamic_slice_in_dim(emb, idx[k], 1, 0) for k in range(_NIDX)],
      axis=0)
  z = rows[0, 0] * 0.0
  npad = _NB * _BV - _VOCAB
  wt = (jnp.pad(W.T + z, ((0, 0), (0, npad)))
        .reshape(_D, _NB, _BV).transpose(1, 0, 2))

  out = pl.pallas_call(
      _fused_body,
      grid=(2, _NB),
      in_specs=[
          pl.BlockSpec((_NIDX, _D), lambda p, i: (0, 0)),
          pl.BlockSpec((1, _D, _BV), lambda p, i: (i * (1 - p), 0, 0)),
          pl.BlockSpec((_BV,), lambda p, i: (i * (1 - p),)),
      ],
      out_specs=pl.BlockSpec((_BV,), lambda p, i: (i * p,)),
      out_shape=jax.ShapeDtypeStruct((_VOCAB,), jnp.float32),
      scratch_shapes=[
          pltpu.VMEM((1, _NB * _BV), jnp.float32),
          pltpu.VMEM((1, _D), jnp.float32),
          pltpu.VMEM((1, 128), jnp.float32),
          pltpu.VMEM((1, 128), jnp.float32),
      ],
      compiler_params=pltpu.CompilerParams(
          vmem_limit_bytes=110 * 1024 * 1024,
      ),
  )(rows, wt, b)

  return out.reshape(1, _VOCAB)


# speedup vs baseline: 2.7322x; 2.7322x over previous
import jax
import jax.numpy as jnp
from jax import lax
from jax.experimental import pallas as pl
from jax.experimental.pallas import tpu as pltpu

_VOCAB = 1_000_000
_D = 64
_NIDX = 20

_BV = 65536
_NB = (_VOCAB + _BV - 1) // _BV
_NEG = -1e30


def _fused_body(rows_ref, w_ref, b_ref, out_ref, scratch, s_ref, m_ref, l_ref):
  p = pl.program_id(0)
  i = pl.program_id(1)

  @pl.when(jnp.logical_and(p == 0, i == 0))
  def _():
    s_ref[...] = jnp.sum(rows_ref[...], axis=0, keepdims=True)
    m_ref[...] = jnp.full((1, 128), _NEG, jnp.float32)
    l_ref[...] = jnp.zeros((1, 128), jnp.float32)

  @pl.when(p == 0)
  def _():
    logits = lax.dot_general(
        s_ref[...], w_ref[...].astype(jnp.float32), (((1,), (0,)), ((), ())),
        preferred_element_type=jnp.float32,
    ) + b_ref[...].reshape(1, _BV)
    vidx = lax.broadcasted_iota(jnp.int32, (1, _BV), 1) + i * _BV
    logits = jnp.where(vidx < _VOCAB, logits, _NEG)
    scratch[:, pl.ds(i * _BV, _BV)] = logits
    m_old = m_ref[...]
    bmax = jnp.max(logits, axis=1, keepdims=True)
    m_new = jnp.maximum(m_old, bmax)
    corr = jnp.exp(m_old - m_new)
    bsum = jnp.sum(jnp.exp(logits - m_new[:, 0:1]), axis=1, keepdims=True)
    l_ref[...] = l_ref[...] * corr + bsum
    m_ref[...] = m_new

  @pl.when(p == 1)
  def _():
    logz = m_ref[...] + jnp.log(l_ref[...])
    res = scratch[:, pl.ds(i * _BV, _BV)] - logz[:, 0:1]
    out_ref[...] = res.reshape(_BV)


def kernel(inputs, emb, W, b):
  idx = inputs.astype(jnp.int32)
  rows = jnp.concatenate(
      [lax.dynamic_slice_in_dim(emb, idx[k], 1, 0) for k in range(_NIDX)],
      axis=0)
  z = rows[0, 0] * 0.0
  wt = (W.T + z).astype(jnp.bfloat16)

  out = pl.pallas_call(
      _fused_body,
      grid=(2, _NB),
      in_specs=[
          pl.BlockSpec((_NIDX, _D), lambda p, i: (0, 0)),
          pl.BlockSpec((_D, _BV), lambda p, i: (0, i * (1 - p))),
          pl.BlockSpec((_BV,), lambda p, i: (i * (1 - p),)),
      ],
      out_specs=pl.BlockSpec((_BV,), lambda p, i: (i * p,)),
      out_shape=jax.ShapeDtypeStruct((_VOCAB,), jnp.float32),
      scratch_shapes=[
          pltpu.VMEM((1, _NB * _BV), jnp.float32),
          pltpu.VMEM((1, _D), jnp.float32),
          pltpu.VMEM((1, 128), jnp.float32),
          pltpu.VMEM((1, 128), jnp.float32),
      ],
      compiler_params=pltpu.CompilerParams(
          vmem_limit_bytes=110 * 1024 * 1024,
      ),
  )(rows, wt, b)

  return out.reshape(1, _VOCAB)
